# baseline (speedup 1.0000x reference)
import dataclasses

import numpy as np
import jax
import jax.numpy as jnp
from jax import lax
from jax.experimental import pallas as pl
from jax.experimental.pallas import tpu as pltpu
from jax.experimental.pallas import tpu_sc as plsc

N = 10000
E = 320000
D = 128
NUM_TYPES = 4

NC = 2
NS = 16
NW = NC * NS

K = 128
CHUNKS = 80
OCT = 8
OCTS = CHUNKS // OCT
EPAD = NW * K * CHUNKS
EPT = E // NW
NPADROWS = 16
NACC = 10112
ROWS_PER_TILE = NACC // NS

_mesh = plsc.VectorSubcoreMesh(core_axis_name="c", subcore_axis_name="s")

_sc_params = pltpu.CompilerParams()
if "needs_layout_passes" in pltpu.CompilerParams.__dataclass_fields__:
    _sc_params = dataclasses.replace(_sc_params, needs_layout_passes=False)


def _deg_body(dst_hbm, deg_hbm, dstbuf, deg_local, sem):
    c = lax.axis_index("c")
    s = lax.axis_index("s")
    wid = s * NC + c
    zero16 = jnp.zeros((16,), jnp.float32)

    @pl.loop(0, N, step=16)
    def _(i):
        deg_local[pl.ds(i, 16)] = zero16

    pltpu.async_copy(dst_hbm.at[wid], dstbuf, sem).wait()
    ones16 = jnp.ones((16,), jnp.float32)

    @pl.loop(0, EPT, step=16)
    def _(e):
        idx = dstbuf[pl.ds(e, 16)]
        plsc.addupdate_scatter(deg_local, [idx], ones16)

    pltpu.sync_copy(deg_local, deg_hbm.at[wid])


@jax.jit
def _sc_degree(dst_r):
    return pl.kernel(
        _deg_body,
        out_type=jax.ShapeDtypeStruct((NW, N), jnp.float32),
        mesh=_mesh,
        scratch_types=[
            pltpu.VMEM((EPT,), jnp.int32),
            pltpu.VMEM((N,), jnp.float32),
            pltpu.SemaphoreType.DMA,
        ],
        compiler_params=_sc_params,
    )(dst_r)


def _agg_body(g_hbm, src_hbm, dst_hbm, out_hbm,
              srcring, dstbuf, rows0, rows1, acc,
              sem_idx, semg0, semg1, sems0, sems1):
    c = lax.axis_index("c")
    s = lax.axis_index("s")
    wid = s * NC + c
    rslice = pl.ds(s * ROWS_PER_TILE, ROWS_PER_TILE)
    src_slab = src_hbm.at[wid]

    idx_cp = pltpu.async_copy(dst_hbm.at[wid], dstbuf, sem_idx)

    @pl.when(s < NS - 1)
    def _():
        sl = pl.ds(s * ROWS_PER_TILE, ROWS_PER_TILE)
        pltpu.sync_copy(g_hbm.at[sl], acc.at[sl])

    @pl.when(s == NS - 1)
    def _():
        sl = pl.ds((NS - 1) * ROWS_PER_TILE, N - (NS - 1) * ROWS_PER_TILE)
        pltpu.sync_copy(g_hbm.at[sl], acc.at[sl])

    idx_cp.wait()
    pltpu.sync_copy(src_slab.at[pl.ds(0, OCT)], srcring.at[0])
    pltpu.async_copy(src_slab.at[pl.ds(OCT, OCT)], srcring.at[1], sem_idx)
    pltpu.async_copy(g_hbm.at[srcring.at[0].at[0]], rows0, semg0)
    plsc.subcore_barrier()

    @pl.loop(0, OCTS)
    def _(o):
        m = lax.rem(o, 2)
        mn = 1 - m
        for i in range(OCT):
            j = OCT * o + i
            rcur, gcur, scur = ((rows0, semg0, sems0) if i % 2 == 0
                                else (rows1, semg1, sems1))
            rnxt, gnxt, snxt = ((rows1, semg1, sems1) if i % 2 == 0
                                else (rows0, semg0, sems0))

            def _wait_scatter():
                pltpu.make_async_copy(rnxt, acc.at[dstbuf.at[0]], snxt).wait()

            if i == 0:
                @pl.when(o > 0)
                def _():
                    _wait_scatter()
            else:
                _wait_scatter()

            if i < OCT - 1:
                pltpu.async_copy(g_hbm.at[srcring.at[m].at[i + 1]], rnxt, gnxt)
            else:
                @pl.when(o + 1 < OCTS)
                def _():
                    pltpu.make_async_copy(
                        src_slab.at[pl.ds((o + 1) * OCT, OCT)],
                        srcring.at[mn], sem_idx).wait()
                    pltpu.async_copy(
                        g_hbm.at[srcring.at[mn].at[0]], rnxt, gnxt)

                @pl.when(o + 2 < OCTS)
                def _():
                    pltpu.async_copy(
                        src_slab.at[pl.ds((o + 2) * OCT, OCT)],
                        srcring.at[m], sem_idx)

            pltpu.make_async_copy(
                g_hbm.at[srcring.at[m].at[i]], rcur, gcur).wait()
            pltpu.async_copy(rcur, acc.at[dstbuf.at[j]], scur, add=True)

    pltpu.make_async_copy(rows1, acc.at[dstbuf.at[0]], sems1).wait()
    plsc.subcore_barrier()
    pltpu.sync_copy(acc.at[rslice], out_hbm.at[c].at[rslice])


@jax.jit
def _sc_aggregate(g, src_t, dst_t):
    return pl.kernel(
        _agg_body,
        out_type=jax.ShapeDtypeStruct((NC, NACC, D), jnp.float32),
        mesh=_mesh,
        scratch_types=[
            pltpu.VMEM((2, OCT, K), jnp.int32),
            pltpu.VMEM((CHUNKS, K), jnp.int32),
            pltpu.VMEM((K, D), jnp.float32),
            pltpu.VMEM((K, D), jnp.float32),
            pltpu.VMEM_SHARED((NACC, D), jnp.float32),
            pltpu.SemaphoreType.DMA,
            pltpu.SemaphoreType.DMA,
            pltpu.SemaphoreType.DMA,
            pltpu.SemaphoreType.DMA,
            pltpu.SemaphoreType.DMA,
        ],
    )(g, src_t, dst_t)


RB = 1000
NRB = N // RB


def _adapter_body(x_ref, nt_ref, aw_ref, ab_ref, w0_ref, dp_ref,
                  dinv_ref, g_ref):
    x = x_ref[...]
    nt = nt_ref[...]
    res = jnp.zeros((RB, D), jnp.float32)
    for t in range(NUM_TYPES):
        h = jnp.tanh(
            jnp.dot(x, aw_ref[t], preferred_element_type=jnp.float32)
            + ab_ref[t][None, :]
        )
        res = jnp.where(nt == t, h, res)
    dinv = lax.rsqrt(jnp.sum(dp_ref[...], axis=1, keepdims=True) + 1.0)
    dinv_ref[...] = dinv
    g_ref[...] = dinv * jnp.dot(res, w0_ref[...],
                                preferred_element_type=jnp.float32)


@jax.jit
def _tc_adapter(x, nt, aw, ab, w0, dp_t):
    return pl.pallas_call(
        _adapter_body,
        grid=(NRB,),
        in_specs=[
            pl.BlockSpec((RB, D), lambda i: (i, 0)),
            pl.BlockSpec((RB, 1), lambda i: (i, 0)),
            pl.BlockSpec((NUM_TYPES, D, D), lambda i: (0, 0, 0)),
            pl.BlockSpec((NUM_TYPES, D), lambda i: (0, 0)),
            pl.BlockSpec((D, D), lambda i: (0, 0)),
            pl.BlockSpec((RB, NW), lambda i: (i, 0)),
        ],
        out_specs=[
            pl.BlockSpec((RB, 1), lambda i: (i, 0)),
            pl.BlockSpec((RB, D), lambda i: (i, 0)),
        ],
        out_shape=[
            jax.ShapeDtypeStruct((N, 1), jnp.float32),
            jax.ShapeDtypeStruct((N, D), jnp.float32),
        ],
    )(x, nt, aw, ab, w0, dp_t)


def _combine_body(p_ref, g_ref, dinv_ref, w_ref, b_ref, g2_ref):
    dinv = dinv_ref[...]
    out = dinv * (p_ref[0] + p_ref[1] - g_ref[...]) + b_ref[...][None, :]
    g2_ref[...] = dinv * jnp.dot(out, w_ref[...], preferred_element_type=jnp.float32)


@jax.jit
def _tc_combine_matmul(p, g, dinv, w, b):
    return pl.pallas_call(
        _combine_body,
        grid=(NRB,),
        in_specs=[
            pl.BlockSpec((NC, RB, D), lambda i: (0, i, 0)),
            pl.BlockSpec((RB, D), lambda i: (i, 0)),
            pl.BlockSpec((RB, 1), lambda i: (i, 0)),
            pl.BlockSpec((D, D), lambda i: (0, 0)),
            pl.BlockSpec((D,), lambda i: (0,)),
        ],
        out_specs=pl.BlockSpec((RB, D), lambda i: (i, 0)),
        out_shape=jax.ShapeDtypeStruct((N, D), jnp.float32),
    )(p, g, dinv, w, b)


def _final_body(p_ref, g_ref, dinv_ref, b_ref, out_ref):
    out_ref[...] = (
        dinv_ref[...] * (p_ref[0] + p_ref[1] - g_ref[...])
        + b_ref[...][None, :]
    )


@jax.jit
def _tc_final(p, g, dinv, b):
    return pl.pallas_call(
        _final_body,
        grid=(NRB,),
        in_specs=[
            pl.BlockSpec((NC, RB, D), lambda i: (0, i, 0)),
            pl.BlockSpec((RB, D), lambda i: (i, 0)),
            pl.BlockSpec((RB, 1), lambda i: (i, 0)),
            pl.BlockSpec((D,), lambda i: (0,)),
        ],
        out_specs=pl.BlockSpec((RB, D), lambda i: (i, 0)),
        out_shape=jax.ShapeDtypeStruct((N, D), jnp.float32),
    )(p, g, dinv, b)


def kernel(node_feature, node_type, edge_time, edge_index, edge_type,
           adapt_w, adapt_b, gcn_w, gcn_b):
    del edge_time, edge_type
    src = edge_index[0].astype(jnp.int32)
    dst = edge_index[1].astype(jnp.int32)

    dst_r = dst.reshape(NW, EPT)
    pad = EPAD - E
    pad_ar = jnp.arange(pad, dtype=jnp.int32)
    src_t = jnp.concatenate([src, pad_ar % N]).reshape(NW, CHUNKS, K)
    dst_t = jnp.concatenate([dst, N + (pad_ar % NPADROWS)]).reshape(NW, CHUNKS, K)
    nt = node_type.astype(jnp.int32).reshape(N, 1)

    deg_parts = _sc_degree(dst_r)
    dinv, g1 = _tc_adapter(node_feature, nt, adapt_w, adapt_b, gcn_w[0],
                           deg_parts.T)
    p1 = _sc_aggregate(g1, src_t, dst_t)
    g2 = _tc_combine_matmul(p1, g1, dinv, gcn_w[1], gcn_b[0])
    p2 = _sc_aggregate(g2, src_t, dst_t)
    return _tc_final(p2, g2, dinv, gcn_b[1])


# speedup vs baseline: 1.0017x; 1.0017x over previous
import dataclasses

import numpy as np
import jax
import jax.numpy as jnp
from jax import lax
from jax.experimental import pallas as pl
from jax.experimental.pallas import tpu as pltpu
from jax.experimental.pallas import tpu_sc as plsc

N = 10000
E = 320000
D = 128
NUM_TYPES = 4

NC = 2
NS = 16
NW = NC * NS

K = 128
CHUNKS = 80
OCT = 8
OCTS = CHUNKS // OCT
EPAD = NW * K * CHUNKS
EPT = E // NW
NPADROWS = 16
NACC = 10112
ROWS_PER_TILE = NACC // NS

_mesh = plsc.VectorSubcoreMesh(core_axis_name="c", subcore_axis_name="s")

_sc_params = pltpu.CompilerParams()
if "needs_layout_passes" in pltpu.CompilerParams.__dataclass_fields__:
    _sc_params = dataclasses.replace(_sc_params, needs_layout_passes=False)


def _deg_body(dst_hbm, deg_hbm, dstbuf, deg_local, sem):
    c = lax.axis_index("c")
    s = lax.axis_index("s")
    wid = s * NC + c
    zero16 = jnp.zeros((16,), jnp.float32)

    @pl.loop(0, N, step=16)
    def _(i):
        deg_local[pl.ds(i, 16)] = zero16

    pltpu.async_copy(dst_hbm.at[wid], dstbuf, sem).wait()
    ones16 = jnp.ones((16,), jnp.float32)

    @pl.loop(0, EPT, step=16)
    def _(e):
        idx = dstbuf[pl.ds(e, 16)]
        plsc.addupdate_scatter(deg_local, [idx], ones16)

    pltpu.sync_copy(deg_local, deg_hbm.at[wid])


@jax.jit
def _sc_degree(dst_r):
    return pl.kernel(
        _deg_body,
        out_type=jax.ShapeDtypeStruct((NW, N), jnp.float32),
        mesh=_mesh,
        scratch_types=[
            pltpu.VMEM((EPT,), jnp.int32),
            pltpu.VMEM((N,), jnp.float32),
            pltpu.SemaphoreType.DMA,
        ],
        compiler_params=_sc_params,
    )(dst_r)


def _agg_body(g_hbm, src_hbm, dst_hbm, out_hbm,
              srcring, dstbuf, rows0, rows1, acc,
              sem_idx, semg0, semg1, sems0, sems1):
    c = lax.axis_index("c")
    s = lax.axis_index("s")
    wid = s * NC + c
    rslice = pl.ds(s * ROWS_PER_TILE, ROWS_PER_TILE)
    src_slab = src_hbm.at[wid]

    idx_cp = pltpu.async_copy(dst_hbm.at[wid], dstbuf, sem_idx)

    @pl.when(s < NS - 1)
    def _():
        sl = pl.ds(s * ROWS_PER_TILE, ROWS_PER_TILE)
        pltpu.sync_copy(g_hbm.at[sl], acc.at[sl])

    @pl.when(s == NS - 1)
    def _():
        sl = pl.ds((NS - 1) * ROWS_PER_TILE, N - (NS - 1) * ROWS_PER_TILE)
        pltpu.sync_copy(g_hbm.at[sl], acc.at[sl])

    idx_cp.wait()
    pltpu.sync_copy(src_slab.at[pl.ds(0, OCT)], srcring.at[0])
    pltpu.async_copy(src_slab.at[pl.ds(OCT, OCT)], srcring.at[1], sem_idx)
    pltpu.async_copy(g_hbm.at[srcring.at[0].at[0]], rows0, semg0)
    plsc.subcore_barrier()

    @pl.loop(0, OCTS)
    def _(o):
        m = lax.rem(o, 2)
        mn = 1 - m
        for i in range(OCT):
            j = OCT * o + i
            rcur, gcur, scur = ((rows0, semg0, sems0) if i % 2 == 0
                                else (rows1, semg1, sems1))
            rnxt, gnxt, snxt = ((rows1, semg1, sems1) if i % 2 == 0
                                else (rows0, semg0, sems0))

            def _wait_scatter():
                pltpu.make_async_copy(rnxt, acc.at[dstbuf.at[0]], snxt).wait()

            if i == 0:
                @pl.when(o > 0)
                def _():
                    _wait_scatter()
            else:
                _wait_scatter()

            if i < OCT - 1:
                pltpu.async_copy(g_hbm.at[srcring.at[m].at[i + 1]], rnxt, gnxt)
            else:
                @pl.when(o + 1 < OCTS)
                def _():
                    pltpu.make_async_copy(
                        src_slab.at[pl.ds((o + 1) * OCT, OCT)],
                        srcring.at[mn], sem_idx).wait()
                    pltpu.async_copy(
                        g_hbm.at[srcring.at[mn].at[0]], rnxt, gnxt)

                @pl.when(o + 2 < OCTS)
                def _():
                    pltpu.async_copy(
                        src_slab.at[pl.ds((o + 2) * OCT, OCT)],
                        srcring.at[m], sem_idx)

            pltpu.make_async_copy(
                g_hbm.at[srcring.at[m].at[i]], rcur, gcur).wait()
            pltpu.async_copy(rcur, acc.at[dstbuf.at[j]], scur, add=True)

    pltpu.make_async_copy(rows1, acc.at[dstbuf.at[0]], sems1).wait()
    plsc.subcore_barrier()
    pltpu.sync_copy(acc.at[rslice], out_hbm.at[c].at[rslice])


@jax.jit
def _sc_aggregate(g, src_t, dst_t):
    return pl.kernel(
        _agg_body,
        out_type=jax.ShapeDtypeStruct((NC, NACC, D), jnp.float32),
        mesh=_mesh,
        scratch_types=[
            pltpu.VMEM((2, OCT, K), jnp.int32),
            pltpu.VMEM((CHUNKS, K), jnp.int32),
            pltpu.VMEM((K, D), jnp.float32),
            pltpu.VMEM((K, D), jnp.float32),
            pltpu.VMEM_SHARED((NACC, D), jnp.float32),
            pltpu.SemaphoreType.DMA,
            pltpu.SemaphoreType.DMA,
            pltpu.SemaphoreType.DMA,
            pltpu.SemaphoreType.DMA,
            pltpu.SemaphoreType.DMA,
        ],
    )(g, src_t, dst_t)


RB = 2000
NRB = N // RB


def _adapter_body(x_ref, nt_ref, aw_ref, ab_ref, w0_ref, h1_ref):
    x = x_ref[...]
    nt = nt_ref[...]
    res = jnp.zeros((RB, D), jnp.float32)
    for t in range(NUM_TYPES):
        h = jnp.tanh(
            jnp.dot(x, aw_ref[t], preferred_element_type=jnp.float32)
            + ab_ref[t][None, :]
        )
        res = jnp.where(nt == t, h, res)
    h1_ref[...] = jnp.dot(res, w0_ref[...], preferred_element_type=jnp.float32)


@jax.jit
def _tc_adapter(x, nt, aw, ab, w0):
    return pl.pallas_call(
        _adapter_body,
        grid=(NRB,),
        in_specs=[
            pl.BlockSpec((RB, D), lambda i: (i, 0)),
            pl.BlockSpec((RB, 1), lambda i: (i, 0)),
            pl.BlockSpec((NUM_TYPES, D, D), lambda i: (0, 0, 0)),
            pl.BlockSpec((NUM_TYPES, D), lambda i: (0, 0)),
            pl.BlockSpec((D, D), lambda i: (0, 0)),
        ],
        out_specs=pl.BlockSpec((RB, D), lambda i: (i, 0)),
        out_shape=jax.ShapeDtypeStruct((N, D), jnp.float32),
    )(x, nt, aw, ab, w0)


def _scale_body(dp_ref, h1_ref, dinv_ref, g_ref):
    dinv = lax.rsqrt(jnp.sum(dp_ref[...], axis=1, keepdims=True) + 1.0)
    dinv_ref[...] = dinv
    g_ref[...] = dinv * h1_ref[...]


@jax.jit
def _tc_scale(dp_t, h1):
    return pl.pallas_call(
        _scale_body,
        grid=(NRB,),
        in_specs=[
            pl.BlockSpec((RB, NW), lambda i: (i, 0)),
            pl.BlockSpec((RB, D), lambda i: (i, 0)),
        ],
        out_specs=[
            pl.BlockSpec((RB, 1), lambda i: (i, 0)),
            pl.BlockSpec((RB, D), lambda i: (i, 0)),
        ],
        out_shape=[
            jax.ShapeDtypeStruct((N, 1), jnp.float32),
            jax.ShapeDtypeStruct((N, D), jnp.float32),
        ],
    )(dp_t, h1)


def _combine_body(p_ref, g_ref, dinv_ref, w_ref, b_ref, g2_ref):
    dinv = dinv_ref[...]
    out = dinv * (p_ref[0] + p_ref[1] - g_ref[...]) + b_ref[...][None, :]
    g2_ref[...] = dinv * jnp.dot(out, w_ref[...], preferred_element_type=jnp.float32)


@jax.jit
def _tc_combine_matmul(p, g, dinv, w, b):
    return pl.pallas_call(
        _combine_body,
        grid=(NRB,),
        in_specs=[
            pl.BlockSpec((NC, RB, D), lambda i: (0, i, 0)),
            pl.BlockSpec((RB, D), lambda i: (i, 0)),
            pl.BlockSpec((RB, 1), lambda i: (i, 0)),
            pl.BlockSpec((D, D), lambda i: (0, 0)),
            pl.BlockSpec((D,), lambda i: (0,)),
        ],
        out_specs=pl.BlockSpec((RB, D), lambda i: (i, 0)),
        out_shape=jax.ShapeDtypeStruct((N, D), jnp.float32),
    )(p, g, dinv, w, b)


def _final_body(p_ref, g_ref, dinv_ref, b_ref, out_ref):
    out_ref[...] = (
        dinv_ref[...] * (p_ref[0] + p_ref[1] - g_ref[...])
        + b_ref[...][None, :]
    )


@jax.jit
def _tc_final(p, g, dinv, b):
    return pl.pallas_call(
        _final_body,
        grid=(NRB,),
        in_specs=[
            pl.BlockSpec((NC, RB, D), lambda i: (0, i, 0)),
            pl.BlockSpec((RB, D), lambda i: (i, 0)),
            pl.BlockSpec((RB, 1), lambda i: (i, 0)),
            pl.BlockSpec((D,), lambda i: (0,)),
        ],
        out_specs=pl.BlockSpec((RB, D), lambda i: (i, 0)),
        out_shape=jax.ShapeDtypeStruct((N, D), jnp.float32),
    )(p, g, dinv, b)


def kernel(node_feature, node_type, edge_time, edge_index, edge_type,
           adapt_w, adapt_b, gcn_w, gcn_b):
    del edge_time, edge_type
    src = edge_index[0].astype(jnp.int32)
    dst = edge_index[1].astype(jnp.int32)

    dst_r = dst.reshape(NW, EPT)
    pad = EPAD - E
    pad_ar = jnp.arange(pad, dtype=jnp.int32)
    src_t = jnp.concatenate([src, pad_ar % N]).reshape(NW, CHUNKS, K)
    dst_t = jnp.concatenate([dst, N + (pad_ar % NPADROWS)]).reshape(NW, CHUNKS, K)
    nt = node_type.astype(jnp.int32).reshape(N, 1)

    deg_parts = _sc_degree(dst_r)
    h1 = _tc_adapter(node_feature, nt, adapt_w, adapt_b, gcn_w[0])
    dinv, g1 = _tc_scale(deg_parts.T, h1)
    p1 = _sc_aggregate(g1, src_t, dst_t)
    g2 = _tc_combine_matmul(p1, g1, dinv, gcn_w[1], gcn_b[0])
    p2 = _sc_aggregate(g2, src_t, dst_t)
    return _tc_final(p2, g2, dinv, gcn_b[1])


# speedup vs baseline: 1.0224x; 1.0207x over previous
import dataclasses

import numpy as np
import jax
import jax.numpy as jnp
from jax import lax
from jax.experimental import pallas as pl
from jax.experimental.pallas import tpu as pltpu
from jax.experimental.pallas import tpu_sc as plsc

N = 10000
E = 320000
D = 128
NUM_TYPES = 4

NC = 2
NS = 16
NW = NC * NS

K = 128
CHUNKS = 80
OCT = 8
OCTS = CHUNKS // OCT
EPAD = NW * K * CHUNKS
EPT = E // NW
NPADROWS = 16
NACC = 10112
ROWS_PER_TILE = NACC // NS

_mesh = plsc.VectorSubcoreMesh(core_axis_name="c", subcore_axis_name="s")

_sc_params = pltpu.CompilerParams()
if "needs_layout_passes" in pltpu.CompilerParams.__dataclass_fields__:
    _sc_params = dataclasses.replace(_sc_params, needs_layout_passes=False)


def _deg_body(dst_hbm, deg_hbm, dstbuf, deg_local, sem):
    c = lax.axis_index("c")
    s = lax.axis_index("s")
    wid = s * NC + c
    zero16 = jnp.zeros((16,), jnp.float32)

    @pl.loop(0, N, step=16)
    def _(i):
        deg_local[pl.ds(i, 16)] = zero16

    pltpu.async_copy(dst_hbm.at[wid], dstbuf, sem).wait()
    ones16 = jnp.ones((16,), jnp.float32)

    @pl.loop(0, EPT, step=16)
    def _(e):
        idx = dstbuf[pl.ds(e, 16)]
        plsc.addupdate_scatter(deg_local, [idx], ones16)

    pltpu.sync_copy(deg_local, deg_hbm.at[wid])


@jax.jit
def _sc_degree(dst_r):
    return pl.kernel(
        _deg_body,
        out_type=jax.ShapeDtypeStruct((NW, N), jnp.float32),
        mesh=_mesh,
        scratch_types=[
            pltpu.VMEM((EPT,), jnp.int32),
            pltpu.VMEM((N,), jnp.float32),
            pltpu.SemaphoreType.DMA,
        ],
        compiler_params=_sc_params,
    )(dst_r)


def _agg_body(g_hbm, src_hbm, dst_hbm, out_hbm,
              srcring, dstbuf, rows0, rows1, acc,
              sem_idx, semg0, semg1, sems0, sems1):
    c = lax.axis_index("c")
    s = lax.axis_index("s")
    wid = s * NC + c
    rslice = pl.ds(s * ROWS_PER_TILE, ROWS_PER_TILE)
    src_slab = src_hbm.at[wid]

    idx_cp = pltpu.async_copy(dst_hbm.at[wid], dstbuf, sem_idx)

    @pl.when(s < NS - 1)
    def _():
        sl = pl.ds(s * ROWS_PER_TILE, ROWS_PER_TILE)
        pltpu.sync_copy(g_hbm.at[sl], acc.at[sl])

    @pl.when(s == NS - 1)
    def _():
        sl = pl.ds((NS - 1) * ROWS_PER_TILE, N - (NS - 1) * ROWS_PER_TILE)
        pltpu.sync_copy(g_hbm.at[sl], acc.at[sl])

    idx_cp.wait()
    pltpu.sync_copy(src_slab.at[pl.ds(0, OCT)], srcring.at[0])
    pltpu.async_copy(src_slab.at[pl.ds(OCT, OCT)], srcring.at[1], sem_idx)
    pltpu.async_copy(g_hbm.at[srcring.at[0].at[0]], rows0, semg0)
    plsc.subcore_barrier()

    @pl.loop(0, OCTS)
    def _(o):
        m = lax.rem(o, 2)
        mn = 1 - m
        for i in range(OCT):
            j = OCT * o + i
            rcur, gcur, scur = ((rows0, semg0, sems0) if i % 2 == 0
                                else (rows1, semg1, sems1))
            rnxt, gnxt, snxt = ((rows1, semg1, sems1) if i % 2 == 0
                                else (rows0, semg0, sems0))

            def _wait_scatter():
                pltpu.make_async_copy(rnxt, acc.at[dstbuf.at[0]], snxt).wait()

            if i == 0:
                @pl.when(o > 0)
                def _():
                    _wait_scatter()
            else:
                _wait_scatter()

            if i < OCT - 1:
                pltpu.async_copy(g_hbm.at[srcring.at[m].at[i + 1]], rnxt, gnxt)
            else:
                @pl.when(o + 1 < OCTS)
                def _():
                    pltpu.make_async_copy(
                        src_slab.at[pl.ds((o + 1) * OCT, OCT)],
                        srcring.at[mn], sem_idx).wait()
                    pltpu.async_copy(
                        g_hbm.at[srcring.at[mn].at[0]], rnxt, gnxt)

                @pl.when(o + 2 < OCTS)
                def _():
                    pltpu.async_copy(
                        src_slab.at[pl.ds((o + 2) * OCT, OCT)],
                        srcring.at[m], sem_idx)

            pltpu.make_async_copy(
                g_hbm.at[srcring.at[m].at[i]], rcur, gcur).wait()
            pltpu.async_copy(rcur, acc.at[dstbuf.at[j]], scur, add=True)

    pltpu.make_async_copy(rows1, acc.at[dstbuf.at[0]], sems1).wait()
    plsc.subcore_barrier()
    pltpu.sync_copy(acc.at[rslice], out_hbm.at[c].at[rslice])


@jax.jit
def _sc_aggregate(g, src_t, dst_t):
    return pl.kernel(
        _agg_body,
        out_type=jax.ShapeDtypeStruct((NC, NACC, D), jnp.float32),
        mesh=_mesh,
        scratch_types=[
            pltpu.VMEM((2, OCT, K), jnp.int32),
            pltpu.VMEM((CHUNKS, K), jnp.int32),
            pltpu.VMEM((K, D), jnp.float32),
            pltpu.VMEM((K, D), jnp.float32),
            pltpu.VMEM_SHARED((NACC, D), jnp.float32),
            pltpu.SemaphoreType.DMA,
            pltpu.SemaphoreType.DMA,
            pltpu.SemaphoreType.DMA,
            pltpu.SemaphoreType.DMA,
            pltpu.SemaphoreType.DMA,
        ],
    )(g, src_t, dst_t)


RB = 2000
NRB = N // RB


def _adapter_body(x_ref, nt_ref, aw_ref, ab_ref, w0_ref, dp_ref,
                  dinv_ref, g_ref):
    x = x_ref[...]
    nt = nt_ref[...]
    h_all = jnp.tanh(
        jnp.dot(x, aw_ref[...], preferred_element_type=jnp.float32)
        + ab_ref[...]
    )
    res = jnp.zeros((RB, D), jnp.float32)
    for t in range(NUM_TYPES):
        res = jnp.where(nt == t, h_all[:, t * D:(t + 1) * D], res)
    dinv = lax.rsqrt(jnp.sum(dp_ref[...], axis=1, keepdims=True) + 1.0)
    dinv_ref[...] = dinv
    g_ref[...] = dinv * jnp.dot(res, w0_ref[...],
                                preferred_element_type=jnp.float32)


@jax.jit
def _tc_adapter(x, nt, aw_w, ab_w, w0, dp_t):
    return pl.pallas_call(
        _adapter_body,
        grid=(NRB,),
        in_specs=[
            pl.BlockSpec((RB, D), lambda i: (i, 0)),
            pl.BlockSpec((RB, 1), lambda i: (i, 0)),
            pl.BlockSpec((D, NUM_TYPES * D), lambda i: (0, 0)),
            pl.BlockSpec((1, NUM_TYPES * D), lambda i: (0, 0)),
            pl.BlockSpec((D, D), lambda i: (0, 0)),
            pl.BlockSpec((RB, NW), lambda i: (i, 0)),
        ],
        out_specs=[
            pl.BlockSpec((RB, 1), lambda i: (i, 0)),
            pl.BlockSpec((RB, D), lambda i: (i, 0)),
        ],
        out_shape=[
            jax.ShapeDtypeStruct((N, 1), jnp.float32),
            jax.ShapeDtypeStruct((N, D), jnp.float32),
        ],
    )(x, nt, aw_w, ab_w, w0, dp_t)


def _combine_body(p_ref, g_ref, dinv_ref, w_ref, b_ref, g2_ref):
    dinv = dinv_ref[...]
    out = dinv * (p_ref[0] + p_ref[1] - g_ref[...]) + b_ref[...][None, :]
    g2_ref[...] = dinv * jnp.dot(out, w_ref[...], preferred_element_type=jnp.float32)


@jax.jit
def _tc_combine_matmul(p, g, dinv, w, b):
    return pl.pallas_call(
        _combine_body,
        grid=(NRB,),
        in_specs=[
            pl.BlockSpec((NC, RB, D), lambda i: (0, i, 0)),
            pl.BlockSpec((RB, D), lambda i: (i, 0)),
            pl.BlockSpec((RB, 1), lambda i: (i, 0)),
            pl.BlockSpec((D, D), lambda i: (0, 0)),
            pl.BlockSpec((D,), lambda i: (0,)),
        ],
        out_specs=pl.BlockSpec((RB, D), lambda i: (i, 0)),
        out_shape=jax.ShapeDtypeStruct((N, D), jnp.float32),
    )(p, g, dinv, w, b)


def _final_body(p_ref, g_ref, dinv_ref, b_ref, out_ref):
    out_ref[...] = (
        dinv_ref[...] * (p_ref[0] + p_ref[1] - g_ref[...])
        + b_ref[...][None, :]
    )


@jax.jit
def _tc_final(p, g, dinv, b):
    return pl.pallas_call(
        _final_body,
        grid=(NRB,),
        in_specs=[
            pl.BlockSpec((NC, RB, D), lambda i: (0, i, 0)),
            pl.BlockSpec((RB, D), lambda i: (i, 0)),
            pl.BlockSpec((RB, 1), lambda i: (i, 0)),
            pl.BlockSpec((D,), lambda i: (0,)),
        ],
        out_specs=pl.BlockSpec((RB, D), lambda i: (i, 0)),
        out_shape=jax.ShapeDtypeStruct((N, D), jnp.float32),
    )(p, g, dinv, b)


def kernel(node_feature, node_type, edge_time, edge_index, edge_type,
           adapt_w, adapt_b, gcn_w, gcn_b):
    del edge_time, edge_type
    src = edge_index[0].astype(jnp.int32)
    dst = edge_index[1].astype(jnp.int32)

    dst_r = dst.reshape(NW, EPT)
    pad = EPAD - E
    pad_ar = jnp.arange(pad, dtype=jnp.int32)
    src_t = jnp.concatenate([src, pad_ar % N]).reshape(NW, CHUNKS, K)
    dst_t = jnp.concatenate([dst, N + (pad_ar % NPADROWS)]).reshape(NW, CHUNKS, K)
    nt = node_type.astype(jnp.int32).reshape(N, 1)

    aw_w = jnp.transpose(adapt_w, (1, 0, 2)).reshape(D, NUM_TYPES * D)
    ab_w = adapt_b.reshape(1, NUM_TYPES * D)

    deg_parts = _sc_degree(dst_r)
    dinv, g1 = _tc_adapter(node_feature, nt, aw_w, ab_w, gcn_w[0],
                           deg_parts.T)
    p1 = _sc_aggregate(g1, src_t, dst_t)
    g2 = _tc_combine_matmul(p1, g1, dinv, gcn_w[1], gcn_b[0])
    p2 = _sc_aggregate(g2, src_t, dst_t)
    return _tc_final(p2, g2, dinv, gcn_b[1])
